# Initial kernel scaffold; baseline (speedup 1.0000x reference)
import jax
import jax.numpy as jnp
from jax.experimental import pallas as pl


def kernel(word_tensor, table):
    raise NotImplementedError("write your pallas kernel here")


# speedup vs baseline: 1.3062x; 1.3062x over previous
import functools

import jax
import jax.numpy as jnp
from jax import lax
from jax.experimental import pallas as pl
from jax.experimental.pallas import tpu as pltpu
from jax.experimental.pallas import tpu_sc as plsc

VOCAB = 1000000
EMBED_DIM = 32
BATCH = 4096
SEQ_LEN = 200

NC, NS = 2, 16
NW = NC * NS
B = BATCH * SEQ_LEN
PERW = B // NW
GRP = 128
NG = PERW // GRP

_mesh = plsc.VectorSubcoreMesh(core_axis_name="c", subcore_axis_name="s")


@functools.partial(
    pl.kernel,
    out_type=jax.ShapeDtypeStruct((B, EMBED_DIM), jnp.float32),
    mesh=_mesh,
    scratch_types=[
        pltpu.VMEM((NG, GRP), jnp.int32),
        pltpu.VMEM((GRP, EMBED_DIM), jnp.float32),
        pltpu.SemaphoreType.DMA,
    ],
    compiler_params=pltpu.CompilerParams(use_tc_tiling_on_sc=False),
)
def _embed_lookup(idx_hbm, table_hbm, out_hbm, idx_v, rows_v, sem):
    w = lax.axis_index("s") * NC + lax.axis_index("c")
    base = w * PERW
    pltpu.sync_copy(idx_hbm.at[w], idx_v)

    def body(j, carry):
        pltpu.async_copy(table_hbm.at[idx_v.at[j]], rows_v, sem).wait()
        pltpu.sync_copy(rows_v, out_hbm.at[pl.ds(base + j * GRP, GRP)])
        return carry

    lax.fori_loop(0, NG, body, 0)


def kernel(word_tensor, table):
    idx = word_tensor.reshape(NW, NG, GRP)
    out = _embed_lookup(idx, table)
    return out.reshape(BATCH, SEQ_LEN, EMBED_DIM)


# speedup vs baseline: 1.4949x; 1.1444x over previous
import functools

import jax
import jax.numpy as jnp
from jax import lax
from jax.experimental import pallas as pl
from jax.experimental.pallas import tpu as pltpu
from jax.experimental.pallas import tpu_sc as plsc

VOCAB = 1000000
EMBED_DIM = 32
BATCH = 4096
SEQ_LEN = 200

NC, NS = 2, 16
NW = NC * NS
B = BATCH * SEQ_LEN
PERW = B // NW
GRP = 128
NG = PERW // GRP
SG = 10
NSG = NG // SG
SGROWS = SG * GRP

_mesh = plsc.VectorSubcoreMesh(core_axis_name="c", subcore_axis_name="s")


@functools.partial(
    pl.kernel,
    out_type=jax.ShapeDtypeStruct((B, EMBED_DIM), jnp.float32),
    mesh=_mesh,
    scratch_types=[
        pltpu.VMEM((NG, GRP), jnp.int32),
        pltpu.VMEM((2, SGROWS, EMBED_DIM), jnp.float32),
        pltpu.SemaphoreType.DMA,
        pltpu.SemaphoreType.DMA,
        pltpu.SemaphoreType.DMA,
    ],
    compiler_params=pltpu.CompilerParams(use_tc_tiling_on_sc=False),
)
def _embed_lookup(idx_hbm, table_hbm, out_hbm, idx_v, rows_v, gsem, ssem0,
                  ssem1):
    w = lax.axis_index("s") * NC + lax.axis_index("c")
    base = w * PERW
    ssems = (ssem0, ssem1)
    pltpu.sync_copy(idx_hbm.at[w], idx_v)

    def super_group(t, buf, wait_store):
        store = pltpu.make_async_copy(
            rows_v.at[buf], out_hbm.at[pl.ds(base + t * SGROWS, SGROWS)],
            ssems[buf],
        )
        if wait_store:
            store.wait()
        copies = [
            pltpu.async_copy(
                table_hbm.at[idx_v.at[t * SG + g]],
                rows_v.at[buf, pl.ds(g * GRP, GRP)],
                gsem,
            )
            for g in range(SG)
        ]
        for c in copies:
            c.wait()
        store.start()

    def body(t2, carry):
        super_group(2 * t2, 0, wait_store=True)
        super_group(2 * t2 + 1, 1, wait_store=True)
        return carry

    super_group(0, 0, wait_store=False)
    super_group(1, 1, wait_store=False)
    lax.fori_loop(1, NSG // 2, body, 0)
    for buf in range(2):
        pltpu.make_async_copy(
            rows_v.at[buf], out_hbm.at[pl.ds(base, SGROWS)], ssems[buf]
        ).wait()


def kernel(word_tensor, table):
    idx = word_tensor.reshape(NW, NG, GRP)
    out = _embed_lookup(idx, table)
    return out.reshape(BATCH, SEQ_LEN, EMBED_DIM)
